# Initial kernel scaffold; baseline (speedup 1.0000x reference)
import jax
import jax.numpy as jnp
from jax.experimental import pallas as pl


def kernel(detections_t, detections_t1, Wp1, bp1, Wp2, bp2, Wp3, bp3, Wa1, ba1, Wa2, ba2, Wr1, br1, Wr2, br2, omega):
    raise NotImplementedError("write your pallas kernel here")


# speedup vs baseline: 206.8064x; 206.8064x over previous
import functools
import math

import jax
import jax.numpy as jnp
from jax import lax
from jax.experimental import pallas as pl
from jax.experimental.pallas import tpu as pltpu
from jax.experimental.pallas import tpu_sc as plsc

N_T = 1000
N_PAD = 1024
DETP = 8
OSC = 112
OSC_PAD = 128
HID = 192
N_STEPS = 5
DT = 0.01
THRESH = 0.1
TWO_PI = 2.0 * math.pi
EPS = 1e-8
NEG = -3.0e38
BIG = 1 << 30
LANES = 16
PRIV = LANES * N_PAD
NCHUNK = N_PAD // LANES


def _tc_body(dt_ref, dt1_ref, wp1_ref, bp1_ref, wp2_ref, bp2_ref, wp3_ref,
             bp3_ref, wr1_ref, br1_ref, wr2_ref, br2_ref, om_ref,
             sim_ref, msim_ref, midx_ref):
    f32 = jnp.float32

    def mlp_phase(d):
        h = jnp.maximum(jnp.dot(d, wp1_ref[...], preferred_element_type=f32)
                        + bp1_ref[...], 0.0)
        h = jnp.maximum(jnp.dot(h, wp2_ref[...], preferred_element_type=f32)
                        + bp2_ref[...], 0.0)
        return jnp.dot(h, wp3_ref[...], preferred_element_type=f32) + bp3_ref[...]

    ph_t = jnp.mod(mlp_phase(dt_ref[...]), TWO_PI)
    ph_1 = jnp.mod(mlp_phase(dt1_ref[...]), TWO_PI)

    step = om_ref[...] * f32(TWO_PI * DT)
    ph = ph_t
    for _ in range(N_STEPS):
        ph = ph + step
    ph = jnp.mod(ph, TWO_PI)
    r = jnp.dot(jnp.maximum(jnp.dot(ph, wr1_ref[...], preferred_element_type=f32)
                            + br1_ref[...], 0.0),
                wr2_ref[...], preferred_element_type=f32) + br2_ref[...]
    ph = jnp.mod(ph + 0.1 * r, TWO_PI)

    osc_mask = lax.broadcasted_iota(jnp.int32, (1, OSC_PAD), 1) < OSC
    ca = jnp.where(osc_mask, jnp.cos(ph), 0.0)
    sa = jnp.where(osc_mask, jnp.sin(ph), 0.0)
    cb = jnp.where(osc_mask, jnp.cos(ph_1), 0.0)
    sb = jnp.where(osc_mask, jnp.sin(ph_1), 0.0)
    na = jnp.sqrt(jnp.sum(ca * ca + sa * sa, axis=1, keepdims=True)) + EPS
    nb = jnp.sqrt(jnp.sum(cb * cb + sb * sb, axis=1, keepdims=True)) + EPS
    dn = (((1,), (1,)), ((), ()))
    sim = (lax.dot_general(ca / na, cb / nb, dn, preferred_element_type=f32)
           + lax.dot_general(sa / na, sb / nb, dn, preferred_element_type=f32))

    col = lax.broadcasted_iota(jnp.int32, (N_PAD, N_PAD), 1)
    simm = jnp.where(col < N_T, sim, NEG)
    msim = jnp.max(simm, axis=1, keepdims=True)
    midx = jnp.min(jnp.where(simm == msim, col, BIG), axis=1, keepdims=True)
    rowi = lax.broadcasted_iota(jnp.int32, (N_PAD, 1), 0)
    msim = jnp.where(rowi < N_T, msim, NEG)

    sim_ref[...] = sim[:N_T, :N_T]
    msim_ref[...] = msim
    midx_ref[...] = midx


_tc_call = pl.pallas_call(
    _tc_body,
    out_shape=(
        jax.ShapeDtypeStruct((N_T, N_T), jnp.float32),
        jax.ShapeDtypeStruct((N_PAD, 1), jnp.float32),
        jax.ShapeDtypeStruct((N_PAD, 1), jnp.int32),
    ),
)


def _sc_body(msim_hbm, midx_hbm, initf_hbm, initi_hbm, out_hbm,
             sims_v, idxs_v, priv_f, best_f, priv_i, best_i, match_v):
    is0 = jnp.logical_and(lax.axis_index("c") == 0, lax.axis_index("s") == 0)

    @pl.when(is0)
    def _():
        pltpu.sync_copy(msim_hbm, sims_v)
        pltpu.sync_copy(midx_hbm, idxs_v)
        pltpu.sync_copy(initf_hbm, priv_f)
        pltpu.sync_copy(initi_hbm, priv_i)

        lanes = lax.iota(jnp.int32, 16)
        laneoff = lanes * N_PAD

        def pass_a(k, carry):
            off = pl.multiple_of(k * LANES, LANES)
            v = sims_v[pl.ds(off, LANES)]
            ix = idxs_v[pl.ds(off, LANES)]
            addr = laneoff + ix
            cur = plsc.load_gather(priv_f, [addr])
            plsc.store_scatter(priv_f, [addr], jnp.maximum(cur, v))
            return carry
        lax.fori_loop(0, NCHUNK, pass_a, 0)

        def red_a(k, carry):
            off = pl.multiple_of(k * LANES, LANES)
            acc = priv_f[pl.ds(off, LANES)]
            for l in range(1, LANES):
                acc = jnp.maximum(acc, priv_f[pl.ds(off + l * N_PAD, LANES)])
            best_f[pl.ds(off, LANES)] = acc
            return carry
        lax.fori_loop(0, NCHUNK, red_a, 0)

        def pass_b(k, carry):
            off = pl.multiple_of(k * LANES, LANES)
            v = sims_v[pl.ds(off, LANES)]
            ix = idxs_v[pl.ds(off, LANES)]
            bj = plsc.load_gather(best_f, [ix])
            cand = jnp.logical_and(v == bj, v >= THRESH)
            rows = k * LANES + lanes
            addr = laneoff + ix
            cur = plsc.load_gather(priv_i, [addr])
            plsc.store_scatter(priv_i, [addr],
                               jnp.where(cand, jnp.minimum(cur, rows), cur))
            return carry
        lax.fori_loop(0, NCHUNK, pass_b, 0)

        def red_b(k, carry):
            off = pl.multiple_of(k * LANES, LANES)
            acc = priv_i[pl.ds(off, LANES)]
            for l in range(1, LANES):
                acc = jnp.minimum(acc, priv_i[pl.ds(off + l * N_PAD, LANES)])
            best_i[pl.ds(off, LANES)] = acc
            return carry
        lax.fori_loop(0, NCHUNK, red_b, 0)

        def init_m(k, carry):
            off = pl.multiple_of(k * LANES, LANES)
            match_v[pl.ds(off, LANES)] = jnp.full((LANES,), -1, jnp.int32)
            return carry
        lax.fori_loop(0, NCHUNK, init_m, 0)

        def pass_c(k, carry):
            off = pl.multiple_of(k * LANES, LANES)
            w = best_i[pl.ds(off, LANES)]
            m = w < N_PAD
            jv = k * LANES + lanes
            plsc.store_scatter(match_v, [jnp.where(m, w, 0)], jv, mask=m)
            return carry
        lax.fori_loop(0, NCHUNK, pass_c, 0)

        pltpu.sync_copy(match_v, out_hbm)


def _make_sc_match():
    return functools.partial(
        pl.kernel,
        mesh=plsc.VectorSubcoreMesh(core_axis_name="c", subcore_axis_name="s"),
        out_type=jax.ShapeDtypeStruct((N_PAD,), jnp.int32),
        compiler_params=pltpu.CompilerParams(needs_layout_passes=False),
        scratch_types=[
            pltpu.VMEM((N_PAD,), jnp.float32),
            pltpu.VMEM((N_PAD,), jnp.int32),
            pltpu.VMEM((PRIV,), jnp.float32),
            pltpu.VMEM((N_PAD,), jnp.float32),
            pltpu.VMEM((PRIV,), jnp.int32),
            pltpu.VMEM((N_PAD,), jnp.int32),
            pltpu.VMEM((N_PAD,), jnp.int32),
        ],
    )(_sc_body)


def kernel(detections_t, detections_t1, Wp1, bp1, Wp2, bp2, Wp3, bp3,
           Wa1, ba1, Wa2, ba2, Wr1, br1, Wr2, br2, omega):
    f32 = jnp.float32
    d0 = jnp.zeros((N_PAD, DETP), f32).at[:N_T, :4].set(detections_t)
    d1 = jnp.zeros((N_PAD, DETP), f32).at[:N_T, :4].set(detections_t1)
    wp1t = jnp.zeros((DETP, HID), f32).at[:4, :].set(Wp1.T)
    wp2t = Wp2.T
    wp3t = jnp.zeros((HID, OSC_PAD), f32).at[:, :OSC].set(Wp3.T)
    wr1t = jnp.zeros((OSC_PAD, HID), f32).at[:OSC, :].set(Wr1.T)
    wr2t = jnp.zeros((HID, OSC_PAD), f32).at[:, :OSC].set(Wr2.T)
    bp3r = jnp.zeros((1, OSC_PAD), f32).at[0, :OSC].set(bp3)
    br2r = jnp.zeros((1, OSC_PAD), f32).at[0, :OSC].set(br2)
    omr = jnp.zeros((1, OSC_PAD), f32).at[0, :OSC].set(omega)

    sim, msim, midx = _tc_call(d0, d1, wp1t, bp1[None, :], wp2t, bp2[None, :],
                               wp3t, bp3r, wr1t, br1[None, :], wr2t, br2r, omr)

    initf = jnp.full((PRIV,), NEG, f32)
    initi = jnp.full((PRIV,), BIG, jnp.int32)
    matches = _make_sc_match()(msim.reshape(N_PAD), midx.reshape(N_PAD),
                               initf, initi)
    return matches[:N_T], sim


# speedup vs baseline: 230.2094x; 1.1132x over previous
import functools
import math

import jax
import jax.numpy as jnp
from jax import lax
from jax.experimental import pallas as pl
from jax.experimental.pallas import tpu as pltpu
from jax.experimental.pallas import tpu_sc as plsc

N_T = 1000
N_PAD = 1024
DETP = 8
OSC = 112
OSC_PAD = 128
HID = 192
N_STEPS = 5
DT = 0.01
THRESH = 0.1
TWO_PI = 2.0 * math.pi
EPS = 1e-8
NEG = -3.0e38
BIG = 1 << 30
LANES = 16
PRIV = LANES * N_PAD
NCHUNK = N_PAD // LANES


def _tc_body(dt_ref, dt1_ref, wp1_ref, bp1_ref, wp2_ref, bp2_ref, wp3_ref,
             bp3_ref, wr1_ref, br1_ref, wr2_ref, br2_ref, om_ref,
             sim_ref, msim_ref, midx_ref):
    f32 = jnp.float32
    dn = (((1,), (1,)), ((), ()))

    def mm(x, w_ref):
        return lax.dot_general(x, w_ref[...], dn, preferred_element_type=f32)

    def mlp_phase(d):
        h = jnp.maximum(mm(d, wp1_ref) + bp1_ref[...], 0.0)
        h = jnp.maximum(mm(h, wp2_ref) + bp2_ref[...], 0.0)
        return mm(h, wp3_ref) + bp3_ref[...]

    ph_t = jnp.mod(mlp_phase(dt_ref[...]), TWO_PI)
    ph_1 = jnp.mod(mlp_phase(dt1_ref[...]), TWO_PI)

    step = om_ref[...] * f32(TWO_PI * DT)
    ph = ph_t
    for _ in range(N_STEPS):
        ph = ph + step
    ph = jnp.mod(ph, TWO_PI)
    r = mm(jnp.maximum(mm(ph, wr1_ref) + br1_ref[...], 0.0), wr2_ref) + br2_ref[...]
    ph = jnp.mod(ph + 0.1 * r, TWO_PI)

    ca, sa = jnp.cos(ph), jnp.sin(ph)
    cb, sb = jnp.cos(ph_1), jnp.sin(ph_1)
    na = jnp.sqrt(jnp.sum(ca * ca + sa * sa, axis=1, keepdims=True)) + EPS
    nb = jnp.sqrt(jnp.sum(cb * cb + sb * sb, axis=1, keepdims=True)) + EPS
    sim = (lax.dot_general(ca / na, cb / nb, dn, preferred_element_type=f32)
           + lax.dot_general(sa / na, sb / nb, dn, preferred_element_type=f32))

    col = lax.broadcasted_iota(jnp.int32, (N_T, N_T), 1)
    msim = jnp.max(sim, axis=1, keepdims=True)
    midx = jnp.min(jnp.where(sim == msim, col, BIG), axis=1, keepdims=True)

    sim_ref[...] = sim
    msim_ref[0:N_T, :] = msim
    msim_ref[N_T:N_PAD, :] = jnp.full((N_PAD - N_T, 1), NEG, f32)
    midx_ref[0:N_T, :] = midx
    midx_ref[N_T:N_PAD, :] = jnp.zeros((N_PAD - N_T, 1), jnp.int32)


_tc_call = pl.pallas_call(
    _tc_body,
    out_shape=(
        jax.ShapeDtypeStruct((N_T, N_T), jnp.float32),
        jax.ShapeDtypeStruct((N_PAD, 1), jnp.float32),
        jax.ShapeDtypeStruct((N_PAD, 1), jnp.int32),
    ),
)


def _sc_body(msim_hbm, midx_hbm, initf_hbm, initi_hbm, out_hbm,
             sims_v, idxs_v, priv_f, best_f, priv_i, best_i, match_v):
    is0 = jnp.logical_and(lax.axis_index("c") == 0, lax.axis_index("s") == 0)

    @pl.when(is0)
    def _():
        pltpu.sync_copy(msim_hbm, sims_v)
        pltpu.sync_copy(midx_hbm, idxs_v)
        pltpu.sync_copy(initf_hbm, priv_f)
        pltpu.sync_copy(initi_hbm, priv_i)

        lanes = lax.iota(jnp.int32, 16)
        laneoff = lanes * N_PAD

        def pass_a(k, carry):
            off = pl.multiple_of(k * LANES, LANES)
            v = sims_v[pl.ds(off, LANES)]
            ix = idxs_v[pl.ds(off, LANES)]
            addr = laneoff + ix
            cur = plsc.load_gather(priv_f, [addr])
            plsc.store_scatter(priv_f, [addr], jnp.maximum(cur, v))
            return carry
        lax.fori_loop(0, NCHUNK, pass_a, 0)

        def red_a(k, carry):
            off = pl.multiple_of(k * LANES, LANES)
            acc = priv_f[pl.ds(off, LANES)]
            for l in range(1, LANES):
                acc = jnp.maximum(acc, priv_f[pl.ds(off + l * N_PAD, LANES)])
            best_f[pl.ds(off, LANES)] = acc
            return carry
        lax.fori_loop(0, NCHUNK, red_a, 0)

        def pass_b(k, carry):
            off = pl.multiple_of(k * LANES, LANES)
            v = sims_v[pl.ds(off, LANES)]
            ix = idxs_v[pl.ds(off, LANES)]
            bj = plsc.load_gather(best_f, [ix])
            cand = jnp.logical_and(v == bj, v >= THRESH)
            rows = k * LANES + lanes
            addr = laneoff + ix
            cur = plsc.load_gather(priv_i, [addr])
            plsc.store_scatter(priv_i, [addr],
                               jnp.where(cand, jnp.minimum(cur, rows), cur))
            return carry
        lax.fori_loop(0, NCHUNK, pass_b, 0)

        def red_b(k, carry):
            off = pl.multiple_of(k * LANES, LANES)
            acc = priv_i[pl.ds(off, LANES)]
            for l in range(1, LANES):
                acc = jnp.minimum(acc, priv_i[pl.ds(off + l * N_PAD, LANES)])
            best_i[pl.ds(off, LANES)] = acc
            return carry
        lax.fori_loop(0, NCHUNK, red_b, 0)

        def init_m(k, carry):
            off = pl.multiple_of(k * LANES, LANES)
            match_v[pl.ds(off, LANES)] = jnp.full((LANES,), -1, jnp.int32)
            return carry
        lax.fori_loop(0, NCHUNK, init_m, 0)

        def pass_c(k, carry):
            off = pl.multiple_of(k * LANES, LANES)
            w = best_i[pl.ds(off, LANES)]
            m = w < N_PAD
            jv = k * LANES + lanes
            plsc.store_scatter(match_v, [jnp.where(m, w, 0)], jv, mask=m)
            return carry
        lax.fori_loop(0, NCHUNK, pass_c, 0)

        pltpu.sync_copy(match_v, out_hbm)


def _make_sc_match():
    return functools.partial(
        pl.kernel,
        mesh=plsc.VectorSubcoreMesh(core_axis_name="c", subcore_axis_name="s"),
        out_type=jax.ShapeDtypeStruct((N_PAD,), jnp.int32),
        compiler_params=pltpu.CompilerParams(needs_layout_passes=False),
        scratch_types=[
            pltpu.VMEM((N_PAD,), jnp.float32),
            pltpu.VMEM((N_PAD,), jnp.int32),
            pltpu.VMEM((PRIV,), jnp.float32),
            pltpu.VMEM((N_PAD,), jnp.float32),
            pltpu.VMEM((PRIV,), jnp.int32),
            pltpu.VMEM((N_PAD,), jnp.int32),
            pltpu.VMEM((N_PAD,), jnp.int32),
        ],
    )(_sc_body)


def kernel(detections_t, detections_t1, Wp1, bp1, Wp2, bp2, Wp3, bp3,
           Wa1, ba1, Wa2, ba2, Wr1, br1, Wr2, br2, omega):
    f32 = jnp.float32
    sim, msim, midx = _tc_call(detections_t, detections_t1, Wp1, bp1, Wp2, bp2,
                               Wp3, bp3, Wr1, br1, Wr2, br2, omega)

    initf = jnp.full((PRIV,), NEG, f32)
    initi = jnp.full((PRIV,), BIG, jnp.int32)
    matches = _make_sc_match()(msim.reshape(N_PAD), midx.reshape(N_PAD),
                               initf, initi)
    return matches[:N_T], sim


# speedup vs baseline: 242.3935x; 1.0529x over previous
import functools
import math

import jax
import jax.numpy as jnp
from jax import lax
from jax.experimental import pallas as pl
from jax.experimental.pallas import tpu as pltpu
from jax.experimental.pallas import tpu_sc as plsc

N_T = 1000
N_PAD = 1024
DETP = 8
OSC = 112
OSC_PAD = 128
HID = 192
N_STEPS = 5
DT = 0.01
THRESH = 0.1
TWO_PI = 2.0 * math.pi
EPS = 1e-8
NEG = -3.0e38
BIG = 1 << 30
LANES = 16
PRIV = LANES * N_PAD
NCHUNK = N_PAD // LANES


def _tc_body(dt_ref, dt1_ref, wp1_ref, bp1_ref, wp2_ref, bp2_ref, wp3_ref,
             bp3_ref, wr1_ref, br1_ref, wr2_ref, br2_ref, om_ref,
             sim_ref, msim_ref, midx_ref):
    f32 = jnp.float32
    dn = (((1,), (1,)), ((), ()))

    def mm(x, w_ref):
        return lax.dot_general(x, w_ref[...], dn, preferred_element_type=f32)

    def mlp_phase(d):
        h = jnp.maximum(mm(d, wp1_ref) + bp1_ref[...], 0.0)
        h = jnp.maximum(mm(h, wp2_ref) + bp2_ref[...], 0.0)
        return mm(h, wp3_ref) + bp3_ref[...]

    ph_t = jnp.mod(mlp_phase(dt_ref[...]), TWO_PI)
    ph_1 = jnp.mod(mlp_phase(dt1_ref[...]), TWO_PI)

    step = om_ref[...] * f32(TWO_PI * DT)
    ph = ph_t
    for _ in range(N_STEPS):
        ph = ph + step
    ph = jnp.mod(ph, TWO_PI)
    r = mm(jnp.maximum(mm(ph, wr1_ref) + br1_ref[...], 0.0), wr2_ref) + br2_ref[...]
    ph = jnp.mod(ph + 0.1 * r, TWO_PI)

    ca, sa = jnp.cos(ph), jnp.sin(ph)
    cb, sb = jnp.cos(ph_1), jnp.sin(ph_1)
    na = jnp.sqrt(jnp.sum(ca * ca + sa * sa, axis=1, keepdims=True)) + EPS
    nb = jnp.sqrt(jnp.sum(cb * cb + sb * sb, axis=1, keepdims=True)) + EPS
    sim = (lax.dot_general(ca / na, cb / nb, dn, preferred_element_type=f32)
           + lax.dot_general(sa / na, sb / nb, dn, preferred_element_type=f32))

    col = lax.broadcasted_iota(jnp.int32, (N_T, N_T), 1)
    msim = jnp.max(sim, axis=1, keepdims=True)
    midx = jnp.min(jnp.where(sim == msim, col, BIG), axis=1, keepdims=True)

    sim_ref[...] = sim
    msim_ref[0:N_T] = msim[:, 0]
    msim_ref[N_T:N_PAD] = jnp.full((N_PAD - N_T,), NEG, f32)
    midx_ref[0:N_T] = midx[:, 0]
    midx_ref[N_T:N_PAD] = jnp.zeros((N_PAD - N_T,), jnp.int32)


_tc_call = pl.pallas_call(
    _tc_body,
    out_shape=(
        jax.ShapeDtypeStruct((N_T, N_T), jnp.float32),
        jax.ShapeDtypeStruct((N_PAD,), jnp.float32),
        jax.ShapeDtypeStruct((N_PAD,), jnp.int32),
    ),
)


def _sc_body(msim_hbm, midx_hbm, initf_hbm, initi_hbm, out_hbm,
             sims_v, idxs_v, priv_f, best_f, priv_i, best_i, match_v):
    is0 = jnp.logical_and(lax.axis_index("c") == 0, lax.axis_index("s") == 0)

    @pl.when(is0)
    def _():
        pltpu.sync_copy(msim_hbm, sims_v)
        pltpu.sync_copy(midx_hbm, idxs_v)
        pltpu.sync_copy(initf_hbm, priv_f)
        pltpu.sync_copy(initi_hbm, priv_i)

        lanes = lax.iota(jnp.int32, 16)
        laneoff = lanes * N_PAD

        def pass_a(k, carry):
            off = pl.multiple_of(k * LANES, LANES)
            v = sims_v[pl.ds(off, LANES)]
            ix = idxs_v[pl.ds(off, LANES)]
            addr = laneoff + ix
            cur = plsc.load_gather(priv_f, [addr])
            plsc.store_scatter(priv_f, [addr], jnp.maximum(cur, v))
            return carry
        lax.fori_loop(0, NCHUNK, pass_a, 0)

        def red_a(k, carry):
            off = pl.multiple_of(k * LANES, LANES)
            acc = priv_f[pl.ds(off, LANES)]
            for l in range(1, LANES):
                acc = jnp.maximum(acc, priv_f[pl.ds(off + l * N_PAD, LANES)])
            best_f[pl.ds(off, LANES)] = acc
            return carry
        lax.fori_loop(0, NCHUNK, red_a, 0)

        def pass_b(k, carry):
            off = pl.multiple_of(k * LANES, LANES)
            v = sims_v[pl.ds(off, LANES)]
            ix = idxs_v[pl.ds(off, LANES)]
            bj = plsc.load_gather(best_f, [ix])
            cand = jnp.logical_and(v == bj, v >= THRESH)
            rows = k * LANES + lanes
            addr = laneoff + ix
            cur = plsc.load_gather(priv_i, [addr])
            plsc.store_scatter(priv_i, [addr],
                               jnp.where(cand, jnp.minimum(cur, rows), cur))
            return carry
        lax.fori_loop(0, NCHUNK, pass_b, 0)

        def red_b(k, carry):
            off = pl.multiple_of(k * LANES, LANES)
            acc = priv_i[pl.ds(off, LANES)]
            for l in range(1, LANES):
                acc = jnp.minimum(acc, priv_i[pl.ds(off + l * N_PAD, LANES)])
            best_i[pl.ds(off, LANES)] = acc
            return carry
        lax.fori_loop(0, NCHUNK, red_b, 0)

        def init_m(k, carry):
            off = pl.multiple_of(k * LANES, LANES)
            match_v[pl.ds(off, LANES)] = jnp.full((LANES,), -1, jnp.int32)
            return carry
        lax.fori_loop(0, NCHUNK, init_m, 0)

        def pass_c(k, carry):
            off = pl.multiple_of(k * LANES, LANES)
            w = best_i[pl.ds(off, LANES)]
            m = w < N_PAD
            jv = k * LANES + lanes
            plsc.store_scatter(match_v, [jnp.where(m, w, 0)], jv, mask=m)
            return carry
        lax.fori_loop(0, NCHUNK, pass_c, 0)

        pltpu.sync_copy(match_v, out_hbm)


def _make_sc_match():
    return functools.partial(
        pl.kernel,
        mesh=plsc.VectorSubcoreMesh(core_axis_name="c", subcore_axis_name="s"),
        out_type=jax.ShapeDtypeStruct((N_PAD,), jnp.int32),
        compiler_params=pltpu.CompilerParams(needs_layout_passes=False),
        scratch_types=[
            pltpu.VMEM((N_PAD,), jnp.float32),
            pltpu.VMEM((N_PAD,), jnp.int32),
            pltpu.VMEM((PRIV,), jnp.float32),
            pltpu.VMEM((N_PAD,), jnp.float32),
            pltpu.VMEM((PRIV,), jnp.int32),
            pltpu.VMEM((N_PAD,), jnp.int32),
            pltpu.VMEM((N_PAD,), jnp.int32),
        ],
    )(_sc_body)


def kernel(detections_t, detections_t1, Wp1, bp1, Wp2, bp2, Wp3, bp3,
           Wa1, ba1, Wa2, ba2, Wr1, br1, Wr2, br2, omega):
    f32 = jnp.float32
    sim, msim, midx = _tc_call(detections_t, detections_t1, Wp1, bp1, Wp2, bp2,
                               Wp3, bp3, Wr1, br1, Wr2, br2, omega)

    initf = jnp.full((PRIV,), NEG, f32)
    initi = jnp.full((PRIV,), BIG, jnp.int32)
    matches = _make_sc_match()(msim, midx, initf, initi)
    return matches[:N_T], sim


# speedup vs baseline: 250.8224x; 1.0348x over previous
import functools
import math

import jax
import jax.numpy as jnp
from jax import lax
from jax.experimental import pallas as pl
from jax.experimental.pallas import tpu as pltpu
from jax.experimental.pallas import tpu_sc as plsc

N_T = 1000
N_PAD = 1024
DETP = 8
OSC = 112
OSC_PAD = 128
HID = 192
N_STEPS = 5
DT = 0.01
THRESH = 0.1
TWO_PI = 2.0 * math.pi
EPS = 1e-8
NEG = -3.0e38
BIG = 1 << 30
LANES = 16
PRIV = LANES * N_PAD
NCHUNK = N_PAD // LANES


def _tc_body(dt_ref, dt1_ref, wp1_ref, bp1_ref, wp2_ref, bp2_ref, wp3_ref,
             bp3_ref, wr1_ref, br1_ref, wr2_ref, br2_ref, om_ref,
             sim_ref, msim_ref, midx_ref):
    f32 = jnp.float32
    dn = (((1,), (1,)), ((), ()))

    def mm(x, w_ref):
        return lax.dot_general(x, w_ref[...], dn, preferred_element_type=f32)

    def mlp_phase(d):
        h = jnp.maximum(mm(d, wp1_ref) + bp1_ref[...], 0.0)
        h = jnp.maximum(mm(h, wp2_ref) + bp2_ref[...], 0.0)
        return mm(h, wp3_ref) + bp3_ref[...]

    ph_t = jnp.mod(mlp_phase(dt_ref[...]), TWO_PI)
    ph_1 = jnp.mod(mlp_phase(dt1_ref[...]), TWO_PI)

    step = om_ref[...] * f32(TWO_PI * DT)
    ph = ph_t
    for _ in range(N_STEPS):
        ph = ph + step
    ph = jnp.mod(ph, TWO_PI)
    r = mm(jnp.maximum(mm(ph, wr1_ref) + br1_ref[...], 0.0), wr2_ref) + br2_ref[...]
    ph = jnp.mod(ph + 0.1 * r, TWO_PI)

    ca, sa = jnp.cos(ph), jnp.sin(ph)
    cb, sb = jnp.cos(ph_1), jnp.sin(ph_1)
    na = jnp.sqrt(jnp.sum(ca * ca + sa * sa, axis=1, keepdims=True)) + EPS
    nb = jnp.sqrt(jnp.sum(cb * cb + sb * sb, axis=1, keepdims=True)) + EPS
    sim = (lax.dot_general(ca / na, cb / nb, dn, preferred_element_type=f32)
           + lax.dot_general(sa / na, sb / nb, dn, preferred_element_type=f32))

    col = lax.broadcasted_iota(jnp.int32, (N_T, N_T), 1)
    msim = jnp.max(sim, axis=1, keepdims=True)
    midx = jnp.min(jnp.where(sim == msim, col, BIG), axis=1, keepdims=True)

    sim_ref[...] = sim
    msim_ref[0:N_T] = msim[:, 0]
    msim_ref[N_T:N_PAD] = jnp.full((N_PAD - N_T,), NEG, f32)
    midx_ref[0:N_T] = midx[:, 0]
    midx_ref[N_T:N_PAD] = jnp.zeros((N_PAD - N_T,), jnp.int32)


_tc_call = pl.pallas_call(
    _tc_body,
    out_shape=(
        jax.ShapeDtypeStruct((N_T, N_T), jnp.float32),
        jax.ShapeDtypeStruct((N_PAD,), jnp.float32),
        jax.ShapeDtypeStruct((N_PAD,), jnp.int32),
    ),
)


def _sc_body(msim_hbm, midx_hbm, initf_hbm, initi_hbm, out_hbm,
             sims_v, idxs_v, priv_f, best_f, priv_i, best_i, match_v, sem):
    is0 = jnp.logical_and(lax.axis_index("c") == 0, lax.axis_index("s") == 0)
    UNR = 4

    @pl.when(is0)
    def _():
        c1 = pltpu.make_async_copy(msim_hbm, sims_v, sem)
        c2 = pltpu.make_async_copy(midx_hbm, idxs_v, sem)
        c3 = pltpu.make_async_copy(initf_hbm, priv_f, sem)
        c4 = pltpu.make_async_copy(initi_hbm, priv_i, sem)
        c1.start(); c2.start(); c3.start(); c4.start()
        c1.wait(); c2.wait(); c3.wait(); c4.wait()

        lanes = lax.iota(jnp.int32, 16)
        laneoff = lanes * N_PAD

        def pass_a(kk, carry):
            for u in range(UNR):
                off = pl.multiple_of(kk * (LANES * UNR) + u * LANES, LANES)
                v = sims_v[pl.ds(off, LANES)]
                ix = idxs_v[pl.ds(off, LANES)]
                addr = laneoff + ix
                cur = plsc.load_gather(priv_f, [addr])
                plsc.store_scatter(priv_f, [addr], jnp.maximum(cur, v))
            return carry
        lax.fori_loop(0, NCHUNK // UNR, pass_a, 0)

        def red_a(kk, carry):
            for u in range(2):
                off = pl.multiple_of(kk * (LANES * 2) + u * LANES, LANES)
                acc = priv_f[pl.ds(off, LANES)]
                for l in range(1, LANES):
                    acc = jnp.maximum(acc, priv_f[pl.ds(off + l * N_PAD, LANES)])
                best_f[pl.ds(off, LANES)] = acc
            return carry
        lax.fori_loop(0, NCHUNK // 2, red_a, 0)

        def pass_b(kk, carry):
            for u in range(UNR):
                k = kk * UNR + u
                off = pl.multiple_of(k * LANES, LANES)
                v = sims_v[pl.ds(off, LANES)]
                ix = idxs_v[pl.ds(off, LANES)]
                bj = plsc.load_gather(best_f, [ix])
                cand = jnp.logical_and(v == bj, v >= THRESH)
                rows = k * LANES + lanes
                addr = laneoff + ix
                cur = plsc.load_gather(priv_i, [addr])
                plsc.store_scatter(priv_i, [addr],
                                   jnp.where(cand, jnp.minimum(cur, rows), cur))
            return carry
        lax.fori_loop(0, NCHUNK // UNR, pass_b, 0)

        def red_b(kk, carry):
            for u in range(2):
                off = pl.multiple_of(kk * (LANES * 2) + u * LANES, LANES)
                acc = priv_i[pl.ds(off, LANES)]
                for l in range(1, LANES):
                    acc = jnp.minimum(acc, priv_i[pl.ds(off + l * N_PAD, LANES)])
                best_i[pl.ds(off, LANES)] = acc
            return carry
        lax.fori_loop(0, NCHUNK // 2, red_b, 0)

        def pass_c(kk, carry):
            for u in range(UNR):
                k = kk * UNR + u
                off = pl.multiple_of(k * LANES, LANES)
                ix = idxs_v[pl.ds(off, LANES)]
                w = plsc.load_gather(best_i, [ix])
                rows = k * LANES + lanes
                match_v[pl.ds(off, LANES)] = jnp.where(w == rows, ix, -1)
            return carry
        lax.fori_loop(0, NCHUNK // UNR, pass_c, 0)

        pltpu.sync_copy(match_v, out_hbm)


def _make_sc_match():
    return functools.partial(
        pl.kernel,
        mesh=plsc.VectorSubcoreMesh(core_axis_name="c", subcore_axis_name="s"),
        out_type=jax.ShapeDtypeStruct((N_PAD,), jnp.int32),
        compiler_params=pltpu.CompilerParams(needs_layout_passes=False),
        scratch_types=[
            pltpu.VMEM((N_PAD,), jnp.float32),
            pltpu.VMEM((N_PAD,), jnp.int32),
            pltpu.VMEM((PRIV,), jnp.float32),
            pltpu.VMEM((N_PAD,), jnp.float32),
            pltpu.VMEM((PRIV,), jnp.int32),
            pltpu.VMEM((N_PAD,), jnp.int32),
            pltpu.VMEM((N_PAD,), jnp.int32),
            pltpu.SemaphoreType.DMA,
        ],
    )(_sc_body)


def kernel(detections_t, detections_t1, Wp1, bp1, Wp2, bp2, Wp3, bp3,
           Wa1, ba1, Wa2, ba2, Wr1, br1, Wr2, br2, omega):
    f32 = jnp.float32
    sim, msim, midx = _tc_call(detections_t, detections_t1, Wp1, bp1, Wp2, bp2,
                               Wp3, bp3, Wr1, br1, Wr2, br2, omega)

    initf = jnp.full((PRIV,), NEG, f32)
    initi = jnp.full((PRIV,), BIG, jnp.int32)
    matches = _make_sc_match()(msim, midx, initf, initi)
    return matches[:N_T], sim


# speedup vs baseline: 255.0338x; 1.0168x over previous
import functools
import math

import jax
import jax.numpy as jnp
from jax import lax
from jax.experimental import pallas as pl
from jax.experimental.pallas import tpu as pltpu
from jax.experimental.pallas import tpu_sc as plsc

N_T = 1000
N_PAD = 1024
DETP = 8
OSC = 112
OSC_PAD = 128
HID = 192
N_STEPS = 5
DT = 0.01
THRESH = 0.1
TWO_PI = 2.0 * math.pi
EPS = 1e-8
NEG = -3.0e38
BIG = 1 << 30
LANES = 16
PRIV = LANES * N_PAD
NCHUNK = N_PAD // LANES


def _tc_body(dt_hbm, dt1_hbm, wp1_hbm, bp1_hbm, wp2_hbm, bp2_hbm, wp3_hbm,
             bp3_hbm, wr1_hbm, br1_hbm, wr2_hbm, br2_hbm, om_hbm,
             sim_ref, msim_ref, midx_ref,
             dt_ref, dt1_ref, wp1_ref, bp1_ref, wp2_ref, bp2_ref, wp3_ref,
             bp3_ref, wr1_ref, br1_ref, wr2_ref, br2_ref, om_ref, sem):
    f32 = jnp.float32
    dn = (((1,), (1,)), ((), ()))

    copies = [pltpu.make_async_copy(h, v, sem) for h, v in (
        (dt_hbm, dt_ref), (dt1_hbm, dt1_ref), (wp1_hbm, wp1_ref),
        (bp1_hbm, bp1_ref), (wp2_hbm, wp2_ref), (bp2_hbm, bp2_ref),
        (wp3_hbm, wp3_ref), (bp3_hbm, bp3_ref), (wr1_hbm, wr1_ref),
        (br1_hbm, br1_ref), (wr2_hbm, wr2_ref), (br2_hbm, br2_ref),
        (om_hbm, om_ref))]
    for c in copies:
        c.start()
    for c in copies:
        c.wait()

    def mm(x, w_ref):
        return lax.dot_general(x, w_ref[...], dn, preferred_element_type=f32)

    def mlp_phase(d):
        h = jnp.maximum(mm(d, wp1_ref) + bp1_ref[...], 0.0)
        h = jnp.maximum(mm(h, wp2_ref) + bp2_ref[...], 0.0)
        return mm(h, wp3_ref) + bp3_ref[...]

    ph_t = jnp.mod(mlp_phase(dt_ref[...]), TWO_PI)
    ph_1 = jnp.mod(mlp_phase(dt1_ref[...]), TWO_PI)

    step = om_ref[...] * f32(TWO_PI * DT)
    ph = ph_t
    for _ in range(N_STEPS):
        ph = ph + step
    ph = jnp.mod(ph, TWO_PI)
    r = mm(jnp.maximum(mm(ph, wr1_ref) + br1_ref[...], 0.0), wr2_ref) + br2_ref[...]
    ph = jnp.mod(ph + 0.1 * r, TWO_PI)

    ca, sa = jnp.cos(ph), jnp.sin(ph)
    cb, sb = jnp.cos(ph_1), jnp.sin(ph_1)
    na = jnp.sqrt(jnp.sum(ca * ca + sa * sa, axis=1, keepdims=True)) + EPS
    nb = jnp.sqrt(jnp.sum(cb * cb + sb * sb, axis=1, keepdims=True)) + EPS
    sim = (lax.dot_general(ca / na, cb / nb, dn, preferred_element_type=f32)
           + lax.dot_general(sa / na, sb / nb, dn, preferred_element_type=f32))

    col = lax.broadcasted_iota(jnp.int32, (N_T, N_T), 1)
    msim = jnp.max(sim, axis=1, keepdims=True)
    midx = jnp.min(jnp.where(sim == msim, col, BIG), axis=1, keepdims=True)

    sim_ref[...] = sim
    msim_ref[0:N_T] = msim[:, 0]
    msim_ref[N_T:N_PAD] = jnp.full((N_PAD - N_T,), NEG, f32)
    midx_ref[0:N_T] = midx[:, 0]
    midx_ref[N_T:N_PAD] = jnp.zeros((N_PAD - N_T,), jnp.int32)


_tc_call = pl.pallas_call(
    _tc_body,
    in_specs=[pl.BlockSpec(memory_space=pl.ANY)] * 13,
    out_shape=(
        jax.ShapeDtypeStruct((N_T, N_T), jnp.float32),
        jax.ShapeDtypeStruct((N_PAD,), jnp.float32),
        jax.ShapeDtypeStruct((N_PAD,), jnp.int32),
    ),
    scratch_shapes=[
        pltpu.VMEM((N_T, 4), jnp.float32),
        pltpu.VMEM((N_T, 4), jnp.float32),
        pltpu.VMEM((HID, 4), jnp.float32),
        pltpu.VMEM((HID,), jnp.float32),
        pltpu.VMEM((HID, HID), jnp.float32),
        pltpu.VMEM((HID,), jnp.float32),
        pltpu.VMEM((OSC, HID), jnp.float32),
        pltpu.VMEM((OSC,), jnp.float32),
        pltpu.VMEM((HID, OSC), jnp.float32),
        pltpu.VMEM((HID,), jnp.float32),
        pltpu.VMEM((OSC, HID), jnp.float32),
        pltpu.VMEM((OSC,), jnp.float32),
        pltpu.VMEM((OSC,), jnp.float32),
        pltpu.SemaphoreType.DMA,
    ],
)


def _sc_body(msim_hbm, midx_hbm, initf_hbm, initi_hbm, out_hbm,
             sims_v, idxs_v, priv_f, best_f, priv_i, best_i, match_v, sem):
    is0 = jnp.logical_and(lax.axis_index("c") == 0, lax.axis_index("s") == 0)
    UNR = 4

    @pl.when(is0)
    def _():
        c1 = pltpu.make_async_copy(msim_hbm, sims_v, sem)
        c2 = pltpu.make_async_copy(midx_hbm, idxs_v, sem)
        c3 = pltpu.make_async_copy(initf_hbm, priv_f, sem)
        c4 = pltpu.make_async_copy(initi_hbm, priv_i, sem)
        c1.start(); c2.start(); c3.start(); c4.start()
        c1.wait(); c2.wait(); c3.wait(); c4.wait()

        lanes = lax.iota(jnp.int32, 16)
        laneoff = lanes * N_PAD

        def pass_a(kk, carry):
            for u in range(UNR):
                off = pl.multiple_of(kk * (LANES * UNR) + u * LANES, LANES)
                v = sims_v[pl.ds(off, LANES)]
                ix = idxs_v[pl.ds(off, LANES)]
                addr = laneoff + ix
                cur = plsc.load_gather(priv_f, [addr])
                plsc.store_scatter(priv_f, [addr], jnp.maximum(cur, v))
            return carry
        lax.fori_loop(0, NCHUNK // UNR, pass_a, 0)

        def red_a(kk, carry):
            for u in range(2):
                off = pl.multiple_of(kk * (LANES * 2) + u * LANES, LANES)
                acc = priv_f[pl.ds(off, LANES)]
                for l in range(1, LANES):
                    acc = jnp.maximum(acc, priv_f[pl.ds(off + l * N_PAD, LANES)])
                best_f[pl.ds(off, LANES)] = acc
            return carry
        lax.fori_loop(0, NCHUNK // 2, red_a, 0)

        def pass_b(kk, carry):
            for u in range(UNR):
                k = kk * UNR + u
                off = pl.multiple_of(k * LANES, LANES)
                v = sims_v[pl.ds(off, LANES)]
                ix = idxs_v[pl.ds(off, LANES)]
                bj = plsc.load_gather(best_f, [ix])
                cand = jnp.logical_and(v == bj, v >= THRESH)
                rows = k * LANES + lanes
                addr = laneoff + ix
                cur = plsc.load_gather(priv_i, [addr])
                plsc.store_scatter(priv_i, [addr],
                                   jnp.where(cand, jnp.minimum(cur, rows), cur))
            return carry
        lax.fori_loop(0, NCHUNK // UNR, pass_b, 0)

        def red_b(kk, carry):
            for u in range(2):
                off = pl.multiple_of(kk * (LANES * 2) + u * LANES, LANES)
                acc = priv_i[pl.ds(off, LANES)]
                for l in range(1, LANES):
                    acc = jnp.minimum(acc, priv_i[pl.ds(off + l * N_PAD, LANES)])
                best_i[pl.ds(off, LANES)] = acc
            return carry
        lax.fori_loop(0, NCHUNK // 2, red_b, 0)

        def pass_c(kk, carry):
            for u in range(UNR):
                k = kk * UNR + u
                off = pl.multiple_of(k * LANES, LANES)
                ix = idxs_v[pl.ds(off, LANES)]
                w = plsc.load_gather(best_i, [ix])
                rows = k * LANES + lanes
                match_v[pl.ds(off, LANES)] = jnp.where(w == rows, ix, -1)
            return carry
        lax.fori_loop(0, NCHUNK // UNR, pass_c, 0)

        pltpu.sync_copy(match_v, out_hbm)


def _make_sc_match():
    return functools.partial(
        pl.kernel,
        mesh=plsc.VectorSubcoreMesh(core_axis_name="c", subcore_axis_name="s"),
        out_type=jax.ShapeDtypeStruct((N_PAD,), jnp.int32),
        compiler_params=pltpu.CompilerParams(needs_layout_passes=False),
        scratch_types=[
            pltpu.VMEM((N_PAD,), jnp.float32),
            pltpu.VMEM((N_PAD,), jnp.int32),
            pltpu.VMEM((PRIV,), jnp.float32),
            pltpu.VMEM((N_PAD,), jnp.float32),
            pltpu.VMEM((PRIV,), jnp.int32),
            pltpu.VMEM((N_PAD,), jnp.int32),
            pltpu.VMEM((N_PAD,), jnp.int32),
            pltpu.SemaphoreType.DMA,
        ],
    )(_sc_body)


def kernel(detections_t, detections_t1, Wp1, bp1, Wp2, bp2, Wp3, bp3,
           Wa1, ba1, Wa2, ba2, Wr1, br1, Wr2, br2, omega):
    f32 = jnp.float32
    sim, msim, midx = _tc_call(detections_t, detections_t1, Wp1, bp1, Wp2, bp2,
                               Wp3, bp3, Wr1, br1, Wr2, br2, omega)

    initf = jnp.full((PRIV,), NEG, f32)
    initi = jnp.full((PRIV,), BIG, jnp.int32)
    matches = _make_sc_match()(msim, midx, initf, initi)
    return matches[:N_T], sim


# speedup vs baseline: 313.7939x; 1.2304x over previous
import functools
import math

import jax
import jax.numpy as jnp
from jax import lax
from jax.experimental import pallas as pl
from jax.experimental.pallas import tpu as pltpu
from jax.experimental.pallas import tpu_sc as plsc

N_T = 1000
N_PAD = 1024
DETP = 8
OSC = 112
OSC_PAD = 128
HID = 192
N_STEPS = 5
DT = 0.01
THRESH = 0.1
TWO_PI = 2.0 * math.pi
EPS = 1e-8
NEG = -3.0e38
BIG = 1 << 30
LANES = 16
PRIV = LANES * N_PAD
NCHUNK = N_PAD // LANES


def _tc_body(dt_hbm, dt1_hbm, wp1_hbm, bp1_hbm, wp2_hbm, bp2_hbm, wp3_hbm,
             bp3_hbm, wr1_hbm, br1_hbm, wr2_hbm, br2_hbm, om_hbm,
             sim_ref, msim_ref, midx_ref,
             dt_ref, dt1_ref, wp1_ref, bp1_ref, wp2_ref, bp2_ref, wp3_ref,
             bp3_ref, wr1_ref, br1_ref, wr2_ref, br2_ref, om_ref, sem):
    f32 = jnp.float32
    cn = (((0,), (0,)), ((), ()))
    sn = (((1,), (0,)), ((), ()))

    copies = [pltpu.make_async_copy(h, v, sem) for h, v in (
        (dt_hbm, dt_ref), (dt1_hbm, dt1_ref), (wp1_hbm, wp1_ref),
        (bp1_hbm, bp1_ref), (wp2_hbm, wp2_ref), (bp2_hbm, bp2_ref),
        (wp3_hbm, wp3_ref), (bp3_hbm, bp3_ref), (wr1_hbm, wr1_ref),
        (br1_hbm, br1_ref), (wr2_hbm, wr2_ref), (br2_hbm, br2_ref),
        (om_hbm, om_ref))]
    for c in copies:
        c.start()
    for c in copies:
        c.wait()

    bp1c = bp1_ref[...][:, None]
    bp2c = bp2_ref[...][:, None]
    bp3c = bp3_ref[...][:, None]
    br1c = br1_ref[...][:, None]
    br2c = br2_ref[...][:, None]

    def mlp_phase(dT):
        h = jnp.maximum(
            lax.dot_general(wp1_ref[...], dT, cn, preferred_element_type=f32)
            + bp1c, 0.0)
        h = jnp.maximum(
            lax.dot_general(wp2_ref[...], h, sn, preferred_element_type=f32)
            + bp2c, 0.0)
        return lax.dot_general(wp3_ref[...], h, cn,
                               preferred_element_type=f32) + bp3c

    ph_t = jnp.mod(mlp_phase(dt_ref[...]), TWO_PI)
    ph_1 = jnp.mod(mlp_phase(dt1_ref[...]), TWO_PI)

    step = om_ref[...][:, None] * f32(TWO_PI * DT)
    ph = ph_t
    for _ in range(N_STEPS):
        ph = ph + step
    ph = jnp.mod(ph, TWO_PI)
    hr = jnp.maximum(
        lax.dot_general(wr1_ref[...], ph, sn, preferred_element_type=f32)
        + br1c, 0.0)
    r = lax.dot_general(wr2_ref[...], hr, cn, preferred_element_type=f32) + br2c
    ph = jnp.mod(ph + 0.1 * r, TWO_PI)

    ca, sa = jnp.cos(ph), jnp.sin(ph)
    cb, sb = jnp.cos(ph_1), jnp.sin(ph_1)
    na = jnp.sqrt(jnp.sum(ca * ca + sa * sa, axis=0, keepdims=True)) + EPS
    nb = jnp.sqrt(jnp.sum(cb * cb + sb * sb, axis=0, keepdims=True)) + EPS
    sim = (lax.dot_general(ca / na, cb / nb, cn, preferred_element_type=f32)
           + lax.dot_general(sa / na, sb / nb, cn, preferred_element_type=f32))

    col = lax.broadcasted_iota(jnp.int32, (N_T, N_T), 1)
    msim = jnp.max(sim, axis=1, keepdims=True)
    midx = jnp.min(jnp.where(sim == msim, col, BIG), axis=1, keepdims=True)

    sim_ref[...] = sim
    msim_ref[0:N_T] = msim[:, 0]
    msim_ref[N_T:N_PAD] = jnp.full((N_PAD - N_T,), NEG, f32)
    midx_ref[0:N_T] = midx[:, 0]
    midx_ref[N_T:N_PAD] = jnp.zeros((N_PAD - N_T,), jnp.int32)


_tc_call = pl.pallas_call(
    _tc_body,
    in_specs=[pl.BlockSpec(memory_space=pl.ANY)] * 13,
    out_shape=(
        jax.ShapeDtypeStruct((N_T, N_T), jnp.float32),
        jax.ShapeDtypeStruct((N_PAD,), jnp.float32),
        jax.ShapeDtypeStruct((N_PAD,), jnp.int32),
    ),
    scratch_shapes=[
        pltpu.VMEM((4, N_T), jnp.float32),
        pltpu.VMEM((4, N_T), jnp.float32),
        pltpu.VMEM((4, HID), jnp.float32),
        pltpu.VMEM((HID,), jnp.float32),
        pltpu.VMEM((HID, HID), jnp.float32),
        pltpu.VMEM((HID,), jnp.float32),
        pltpu.VMEM((HID, OSC), jnp.float32),
        pltpu.VMEM((OSC,), jnp.float32),
        pltpu.VMEM((HID, OSC), jnp.float32),
        pltpu.VMEM((HID,), jnp.float32),
        pltpu.VMEM((HID, OSC), jnp.float32),
        pltpu.VMEM((OSC,), jnp.float32),
        pltpu.VMEM((OSC,), jnp.float32),
        pltpu.SemaphoreType.DMA,
    ],
)


def _sc_body(msim_hbm, midx_hbm, initf_hbm, initi_hbm, out_hbm,
             sims_v, idxs_v, priv_f, best_f, priv_i, best_i, match_v, sem):
    is0 = jnp.logical_and(lax.axis_index("c") == 0, lax.axis_index("s") == 0)
    UNR = 4

    @pl.when(is0)
    def _():
        c1 = pltpu.make_async_copy(msim_hbm, sims_v, sem)
        c2 = pltpu.make_async_copy(midx_hbm, idxs_v, sem)
        c3 = pltpu.make_async_copy(initf_hbm, priv_f, sem)
        c4 = pltpu.make_async_copy(initi_hbm, priv_i, sem)
        c1.start(); c2.start(); c3.start(); c4.start()
        c1.wait(); c2.wait(); c3.wait(); c4.wait()

        lanes = lax.iota(jnp.int32, 16)
        laneoff = lanes * N_PAD

        def pass_a(kk, carry):
            for u in range(UNR):
                off = pl.multiple_of(kk * (LANES * UNR) + u * LANES, LANES)
                v = sims_v[pl.ds(off, LANES)]
                ix = idxs_v[pl.ds(off, LANES)]
                addr = laneoff + ix
                cur = plsc.load_gather(priv_f, [addr])
                plsc.store_scatter(priv_f, [addr], jnp.maximum(cur, v))
            return carry
        lax.fori_loop(0, NCHUNK // UNR, pass_a, 0)

        def red_a(kk, carry):
            for u in range(2):
                off = pl.multiple_of(kk * (LANES * 2) + u * LANES, LANES)
                acc = priv_f[pl.ds(off, LANES)]
                for l in range(1, LANES):
                    acc = jnp.maximum(acc, priv_f[pl.ds(off + l * N_PAD, LANES)])
                best_f[pl.ds(off, LANES)] = acc
            return carry
        lax.fori_loop(0, NCHUNK // 2, red_a, 0)

        def pass_b(kk, carry):
            for u in range(UNR):
                k = kk * UNR + u
                off = pl.multiple_of(k * LANES, LANES)
                v = sims_v[pl.ds(off, LANES)]
                ix = idxs_v[pl.ds(off, LANES)]
                bj = plsc.load_gather(best_f, [ix])
                cand = jnp.logical_and(v == bj, v >= THRESH)
                rows = k * LANES + lanes
                addr = laneoff + ix
                cur = plsc.load_gather(priv_i, [addr])
                plsc.store_scatter(priv_i, [addr],
                                   jnp.where(cand, jnp.minimum(cur, rows), cur))
            return carry
        lax.fori_loop(0, NCHUNK // UNR, pass_b, 0)

        def red_b(kk, carry):
            for u in range(2):
                off = pl.multiple_of(kk * (LANES * 2) + u * LANES, LANES)
                acc = priv_i[pl.ds(off, LANES)]
                for l in range(1, LANES):
                    acc = jnp.minimum(acc, priv_i[pl.ds(off + l * N_PAD, LANES)])
                best_i[pl.ds(off, LANES)] = acc
            return carry
        lax.fori_loop(0, NCHUNK // 2, red_b, 0)

        def pass_c(kk, carry):
            for u in range(UNR):
                k = kk * UNR + u
                off = pl.multiple_of(k * LANES, LANES)
                ix = idxs_v[pl.ds(off, LANES)]
                w = plsc.load_gather(best_i, [ix])
                rows = k * LANES + lanes
                match_v[pl.ds(off, LANES)] = jnp.where(w == rows, ix, -1)
            return carry
        lax.fori_loop(0, NCHUNK // UNR, pass_c, 0)

        pltpu.sync_copy(match_v, out_hbm)


def _make_sc_match():
    return functools.partial(
        pl.kernel,
        mesh=plsc.VectorSubcoreMesh(core_axis_name="c", subcore_axis_name="s"),
        out_type=jax.ShapeDtypeStruct((N_PAD,), jnp.int32),
        compiler_params=pltpu.CompilerParams(needs_layout_passes=False),
        scratch_types=[
            pltpu.VMEM((N_PAD,), jnp.float32),
            pltpu.VMEM((N_PAD,), jnp.int32),
            pltpu.VMEM((PRIV,), jnp.float32),
            pltpu.VMEM((N_PAD,), jnp.float32),
            pltpu.VMEM((PRIV,), jnp.int32),
            pltpu.VMEM((N_PAD,), jnp.int32),
            pltpu.VMEM((N_PAD,), jnp.int32),
            pltpu.SemaphoreType.DMA,
        ],
    )(_sc_body)


def kernel(detections_t, detections_t1, Wp1, bp1, Wp2, bp2, Wp3, bp3,
           Wa1, ba1, Wa2, ba2, Wr1, br1, Wr2, br2, omega):
    f32 = jnp.float32
    sim, msim, midx = _tc_call(detections_t.T, detections_t1.T, Wp1.T, bp1,
                               Wp2, bp2, Wp3.T, bp3, Wr1, br1, Wr2.T, br2,
                               omega)

    initf = jnp.full((PRIV,), NEG, f32)
    initi = jnp.full((PRIV,), BIG, jnp.int32)
    matches = _make_sc_match()(msim, midx, initf, initi)
    return matches[:N_T], sim


# speedup vs baseline: 315.2393x; 1.0046x over previous
import functools
import math

import jax
import jax.numpy as jnp
from jax import lax
from jax.experimental import pallas as pl
from jax.experimental.pallas import tpu as pltpu
from jax.experimental.pallas import tpu_sc as plsc

N_T = 1000
N_PAD = 1024
DETP = 8
OSC = 112
OSC_PAD = 128
HID = 192
N_STEPS = 5
DT = 0.01
THRESH = 0.1
TWO_PI = 2.0 * math.pi
EPS = 1e-8
NEG = -3.0e38
BIG = 1 << 30
LANES = 16
PRIV = LANES * N_PAD
NCHUNK = N_PAD // LANES


def _tc_body(dt_hbm, dt1_hbm, wp1_hbm, bp1_hbm, wp2_hbm, bp2_hbm, wp3_hbm,
             bp3_hbm, wr1_hbm, br1_hbm, wr2_hbm, br2_hbm, om_hbm,
             sim_ref, msim_ref, midx_ref,
             dt_ref, dt1_ref, wp1_ref, bp1_ref, wp2_ref, bp2_ref, wp3_ref,
             bp3_ref, wr1_ref, br1_ref, wr2_ref, br2_ref, om_ref, sem):
    f32 = jnp.float32
    cn = (((0,), (0,)), ((), ()))
    sn = (((1,), (0,)), ((), ()))

    copies = [pltpu.make_async_copy(h, v, sem) for h, v in (
        (dt_hbm, dt_ref), (dt1_hbm, dt1_ref), (wp1_hbm, wp1_ref),
        (bp1_hbm, bp1_ref), (wp2_hbm, wp2_ref), (bp2_hbm, bp2_ref),
        (wp3_hbm, wp3_ref), (bp3_hbm, bp3_ref), (wr1_hbm, wr1_ref),
        (br1_hbm, br1_ref), (wr2_hbm, wr2_ref), (br2_hbm, br2_ref),
        (om_hbm, om_ref))]
    for c in copies:
        c.start()
    for c in copies:
        c.wait()

    bp1c = bp1_ref[...][:, None]
    bp2c = bp2_ref[...][:, None]
    bp3c = bp3_ref[...][:, None]
    br1c = br1_ref[...][:, None]
    br2c = br2_ref[...][:, None]

    def mlp_phase(dT):
        h = jnp.maximum(
            lax.dot_general(wp1_ref[...], dT, cn, preferred_element_type=f32)
            + bp1c, 0.0)
        h = jnp.maximum(
            lax.dot_general(wp2_ref[...], h, sn, preferred_element_type=f32)
            + bp2c, 0.0)
        return lax.dot_general(wp3_ref[...], h, cn,
                               preferred_element_type=f32) + bp3c

    ph_t = jnp.mod(mlp_phase(dt_ref[...]), TWO_PI)
    ph_1 = jnp.mod(mlp_phase(dt1_ref[...]), TWO_PI)

    step = om_ref[...][:, None] * f32(TWO_PI * DT)
    ph = ph_t
    for _ in range(N_STEPS):
        ph = ph + step
    ph = jnp.mod(ph, TWO_PI)
    hr = jnp.maximum(
        lax.dot_general(wr1_ref[...], ph, sn, preferred_element_type=f32)
        + br1c, 0.0)
    r = lax.dot_general(wr2_ref[...], hr, cn, preferred_element_type=f32) + br2c
    ph = jnp.mod(ph + 0.1 * r, TWO_PI)

    ca, sa = jnp.cos(ph), jnp.sin(ph)
    cb, sb = jnp.cos(ph_1), jnp.sin(ph_1)
    na = jnp.sqrt(jnp.sum(ca * ca + sa * sa, axis=0, keepdims=True)) + EPS
    nb = jnp.sqrt(jnp.sum(cb * cb + sb * sb, axis=0, keepdims=True)) + EPS
    sim = (lax.dot_general(ca / na, cb / nb, cn, preferred_element_type=f32)
           + lax.dot_general(sa / na, sb / nb, cn, preferred_element_type=f32))

    sim_ref[...] = sim

    colf = lax.broadcasted_iota(jnp.int32, (1, N_T), 1).astype(f32)
    msim = jnp.max(sim, axis=1, keepdims=True)
    midxf = jnp.min(jnp.where(sim == msim, colf, f32(3e38)), axis=1,
                    keepdims=True)
    midx = midxf.astype(jnp.int32)

    msim_ref[0:N_T] = msim[:, 0]
    msim_ref[N_T:N_PAD] = jnp.full((N_PAD - N_T,), NEG, f32)
    midx_ref[0:N_T] = midx[:, 0]
    midx_ref[N_T:N_PAD] = jnp.zeros((N_PAD - N_T,), jnp.int32)


_tc_call = pl.pallas_call(
    _tc_body,
    in_specs=[pl.BlockSpec(memory_space=pl.ANY)] * 13,
    out_shape=(
        jax.ShapeDtypeStruct((N_T, N_T), jnp.float32),
        jax.ShapeDtypeStruct((N_PAD,), jnp.float32),
        jax.ShapeDtypeStruct((N_PAD,), jnp.int32),
    ),
    scratch_shapes=[
        pltpu.VMEM((4, N_T), jnp.float32),
        pltpu.VMEM((4, N_T), jnp.float32),
        pltpu.VMEM((4, HID), jnp.float32),
        pltpu.VMEM((HID,), jnp.float32),
        pltpu.VMEM((HID, HID), jnp.float32),
        pltpu.VMEM((HID,), jnp.float32),
        pltpu.VMEM((HID, OSC), jnp.float32),
        pltpu.VMEM((OSC,), jnp.float32),
        pltpu.VMEM((HID, OSC), jnp.float32),
        pltpu.VMEM((HID,), jnp.float32),
        pltpu.VMEM((HID, OSC), jnp.float32),
        pltpu.VMEM((OSC,), jnp.float32),
        pltpu.VMEM((OSC,), jnp.float32),
        pltpu.SemaphoreType.DMA,
    ],
)


def _sc_body(msim_hbm, midx_hbm, initf_hbm, initi_hbm, out_hbm,
             sims_v, idxs_v, priv_f, best_f, priv_i, best_i, match_v, sem):
    is0 = jnp.logical_and(lax.axis_index("c") == 0, lax.axis_index("s") == 0)
    UNR = 4

    @pl.when(is0)
    def _():
        c1 = pltpu.make_async_copy(msim_hbm, sims_v, sem)
        c2 = pltpu.make_async_copy(midx_hbm, idxs_v, sem)
        c3 = pltpu.make_async_copy(initf_hbm, priv_f, sem)
        c4 = pltpu.make_async_copy(initi_hbm, priv_i, sem)
        c1.start(); c2.start(); c3.start(); c4.start()
        c1.wait(); c2.wait(); c3.wait(); c4.wait()

        lanes = lax.iota(jnp.int32, 16)
        laneoff = lanes * N_PAD

        def pass_a(kk, carry):
            for u in range(UNR):
                off = pl.multiple_of(kk * (LANES * UNR) + u * LANES, LANES)
                v = sims_v[pl.ds(off, LANES)]
                ix = idxs_v[pl.ds(off, LANES)]
                addr = laneoff + ix
                cur = plsc.load_gather(priv_f, [addr])
                plsc.store_scatter(priv_f, [addr], jnp.maximum(cur, v))
            return carry
        lax.fori_loop(0, NCHUNK // UNR, pass_a, 0)

        def red_a(kk, carry):
            for u in range(2):
                off = pl.multiple_of(kk * (LANES * 2) + u * LANES, LANES)
                acc = priv_f[pl.ds(off, LANES)]
                for l in range(1, LANES):
                    acc = jnp.maximum(acc, priv_f[pl.ds(off + l * N_PAD, LANES)])
                best_f[pl.ds(off, LANES)] = acc
            return carry
        lax.fori_loop(0, NCHUNK // 2, red_a, 0)

        def pass_b(kk, carry):
            for u in range(UNR):
                k = kk * UNR + u
                off = pl.multiple_of(k * LANES, LANES)
                v = sims_v[pl.ds(off, LANES)]
                ix = idxs_v[pl.ds(off, LANES)]
                bj = plsc.load_gather(best_f, [ix])
                cand = jnp.logical_and(v == bj, v >= THRESH)
                rows = k * LANES + lanes
                addr = laneoff + ix
                cur = plsc.load_gather(priv_i, [addr])
                plsc.store_scatter(priv_i, [addr],
                                   jnp.where(cand, jnp.minimum(cur, rows), cur))
            return carry
        lax.fori_loop(0, NCHUNK // UNR, pass_b, 0)

        def red_b(kk, carry):
            for u in range(2):
                off = pl.multiple_of(kk * (LANES * 2) + u * LANES, LANES)
                acc = priv_i[pl.ds(off, LANES)]
                for l in range(1, LANES):
                    acc = jnp.minimum(acc, priv_i[pl.ds(off + l * N_PAD, LANES)])
                best_i[pl.ds(off, LANES)] = acc
            return carry
        lax.fori_loop(0, NCHUNK // 2, red_b, 0)

        def pass_c(kk, carry):
            for u in range(UNR):
                k = kk * UNR + u
                off = pl.multiple_of(k * LANES, LANES)
                ix = idxs_v[pl.ds(off, LANES)]
                w = plsc.load_gather(best_i, [ix])
                rows = k * LANES + lanes
                match_v[pl.ds(off, LANES)] = jnp.where(w == rows, ix, -1)
            return carry
        lax.fori_loop(0, NCHUNK // UNR, pass_c, 0)

        pltpu.sync_copy(match_v, out_hbm)


def _make_sc_match():
    return functools.partial(
        pl.kernel,
        mesh=plsc.VectorSubcoreMesh(core_axis_name="c", subcore_axis_name="s"),
        out_type=jax.ShapeDtypeStruct((N_PAD,), jnp.int32),
        compiler_params=pltpu.CompilerParams(needs_layout_passes=False),
        scratch_types=[
            pltpu.VMEM((N_PAD,), jnp.float32),
            pltpu.VMEM((N_PAD,), jnp.int32),
            pltpu.VMEM((PRIV,), jnp.float32),
            pltpu.VMEM((N_PAD,), jnp.float32),
            pltpu.VMEM((PRIV,), jnp.int32),
            pltpu.VMEM((N_PAD,), jnp.int32),
            pltpu.VMEM((N_PAD,), jnp.int32),
            pltpu.SemaphoreType.DMA,
        ],
    )(_sc_body)


def kernel(detections_t, detections_t1, Wp1, bp1, Wp2, bp2, Wp3, bp3,
           Wa1, ba1, Wa2, ba2, Wr1, br1, Wr2, br2, omega):
    f32 = jnp.float32
    sim, msim, midx = _tc_call(detections_t.T, detections_t1.T, Wp1.T, bp1,
                               Wp2, bp2, Wp3.T, bp3, Wr1, br1, Wr2.T, br2,
                               omega)

    initf = jnp.full((PRIV,), NEG, f32)
    initi = jnp.full((PRIV,), BIG, jnp.int32)
    matches = _make_sc_match()(msim, midx, initf, initi)
    return matches[:N_T], sim


# speedup vs baseline: 320.1573x; 1.0156x over previous
import functools
import math

import numpy as np

import jax
import jax.numpy as jnp
from jax import lax
from jax.experimental import pallas as pl
from jax.experimental.pallas import tpu as pltpu
from jax.experimental.pallas import tpu_sc as plsc

N_T = 1000
N_PAD = 1024
DETP = 8
OSC = 112
OSC_PAD = 128
HID = 192
N_STEPS = 5
DT = 0.01
THRESH = 0.1
TWO_PI = 2.0 * math.pi
EPS = 1e-8
NEG = -3.0e38
BIG = 1 << 30
LANES = 16
PRIV = LANES * N_PAD
NCHUNK = N_PAD // LANES

_INITF_NP = np.full((PRIV,), NEG, np.float32)
_INITI_NP = np.full((PRIV,), BIG, np.int32)


def _tc_body(dt_hbm, dt1_hbm, wp1_hbm, bp1_hbm, wp2_hbm, bp2_hbm, wp3_hbm,
             bp3_hbm, wr1_hbm, br1_hbm, wr2_hbm, br2_hbm, om_hbm,
             sim_ref, msim_ref, midx_ref,
             dt_ref, dt1_ref, wp1_ref, bp1_ref, wp2_ref, bp2_ref, wp3_ref,
             bp3_ref, wr1_ref, br1_ref, wr2_ref, br2_ref, om_ref, sem):
    f32 = jnp.float32
    cn = (((0,), (0,)), ((), ()))
    sn = (((1,), (0,)), ((), ()))

    copies = [pltpu.make_async_copy(h, v, sem) for h, v in (
        (dt_hbm, dt_ref), (dt1_hbm, dt1_ref), (wp1_hbm, wp1_ref),
        (bp1_hbm, bp1_ref), (wp2_hbm, wp2_ref), (bp2_hbm, bp2_ref),
        (wp3_hbm, wp3_ref), (bp3_hbm, bp3_ref), (wr1_hbm, wr1_ref),
        (br1_hbm, br1_ref), (wr2_hbm, wr2_ref), (br2_hbm, br2_ref),
        (om_hbm, om_ref))]
    for c in copies:
        c.start()
    for c in copies:
        c.wait()

    bp1c = bp1_ref[...][:, None]
    bp2c = bp2_ref[...][:, None]
    bp3c = bp3_ref[...][:, None]
    br1c = br1_ref[...][:, None]
    br2c = br2_ref[...][:, None]

    def mlp_phase(dT):
        h = jnp.maximum(
            lax.dot_general(wp1_ref[...], dT, cn, preferred_element_type=f32)
            + bp1c, 0.0)
        h = jnp.maximum(
            lax.dot_general(wp2_ref[...], h, sn, preferred_element_type=f32)
            + bp2c, 0.0)
        return lax.dot_general(wp3_ref[...], h, cn,
                               preferred_element_type=f32) + bp3c

    ph_t = jnp.mod(mlp_phase(dt_ref[...]), TWO_PI)
    ph_1 = jnp.mod(mlp_phase(dt1_ref[...]), TWO_PI)

    step = om_ref[...][:, None] * f32(TWO_PI * DT)
    ph = ph_t
    for _ in range(N_STEPS):
        ph = ph + step
    ph = jnp.mod(ph, TWO_PI)
    hr = jnp.maximum(
        lax.dot_general(wr1_ref[...], ph, sn, preferred_element_type=f32)
        + br1c, 0.0)
    r = lax.dot_general(wr2_ref[...], hr, cn, preferred_element_type=f32) + br2c
    ph = jnp.mod(ph + 0.1 * r, TWO_PI)

    ca, sa = jnp.cos(ph), jnp.sin(ph)
    cb, sb = jnp.cos(ph_1), jnp.sin(ph_1)
    na = jnp.sqrt(jnp.sum(ca * ca + sa * sa, axis=0, keepdims=True)) + EPS
    nb = jnp.sqrt(jnp.sum(cb * cb + sb * sb, axis=0, keepdims=True)) + EPS
    sim = (lax.dot_general(ca / na, cb / nb, cn, preferred_element_type=f32)
           + lax.dot_general(sa / na, sb / nb, cn, preferred_element_type=f32))

    sim_ref[...] = sim

    colf = lax.broadcasted_iota(jnp.int32, (1, N_T), 1).astype(f32)
    msim = jnp.max(sim, axis=1, keepdims=True)
    midxf = jnp.min(jnp.where(sim == msim, colf, f32(3e38)), axis=1,
                    keepdims=True)
    midx = midxf.astype(jnp.int32)

    msim_ref[0:N_T] = msim[:, 0]
    msim_ref[N_T:N_PAD] = jnp.full((N_PAD - N_T,), NEG, f32)
    midx_ref[0:N_T] = midx[:, 0]
    midx_ref[N_T:N_PAD] = jnp.zeros((N_PAD - N_T,), jnp.int32)


_tc_call = pl.pallas_call(
    _tc_body,
    in_specs=[pl.BlockSpec(memory_space=pl.ANY)] * 13,
    out_shape=(
        jax.ShapeDtypeStruct((N_T, N_T), jnp.float32),
        jax.ShapeDtypeStruct((N_PAD,), jnp.float32),
        jax.ShapeDtypeStruct((N_PAD,), jnp.int32),
    ),
    scratch_shapes=[
        pltpu.VMEM((4, N_T), jnp.float32),
        pltpu.VMEM((4, N_T), jnp.float32),
        pltpu.VMEM((4, HID), jnp.float32),
        pltpu.VMEM((HID,), jnp.float32),
        pltpu.VMEM((HID, HID), jnp.float32),
        pltpu.VMEM((HID,), jnp.float32),
        pltpu.VMEM((HID, OSC), jnp.float32),
        pltpu.VMEM((OSC,), jnp.float32),
        pltpu.VMEM((HID, OSC), jnp.float32),
        pltpu.VMEM((HID,), jnp.float32),
        pltpu.VMEM((HID, OSC), jnp.float32),
        pltpu.VMEM((OSC,), jnp.float32),
        pltpu.VMEM((OSC,), jnp.float32),
        pltpu.SemaphoreType.DMA,
    ],
)


def _sc_body(msim_hbm, midx_hbm, initf_hbm, initi_hbm, out_hbm,
             sims_v, idxs_v, priv_f, best_f, priv_i, best_i, match_v, sem):
    is0 = jnp.logical_and(lax.axis_index("c") == 0, lax.axis_index("s") == 0)
    UNR = 4

    @pl.when(is0)
    def _():
        c1 = pltpu.make_async_copy(msim_hbm, sims_v, sem)
        c2 = pltpu.make_async_copy(midx_hbm, idxs_v, sem)
        c3 = pltpu.make_async_copy(initf_hbm, priv_f, sem)
        c4 = pltpu.make_async_copy(initi_hbm, priv_i, sem)
        c1.start(); c2.start(); c3.start(); c4.start()
        c1.wait(); c2.wait(); c3.wait(); c4.wait()

        lanes = lax.iota(jnp.int32, 16)
        laneoff = lanes * N_PAD

        def pass_a(kk, carry):
            for u in range(UNR):
                off = pl.multiple_of(kk * (LANES * UNR) + u * LANES, LANES)
                v = sims_v[pl.ds(off, LANES)]
                ix = idxs_v[pl.ds(off, LANES)]
                addr = laneoff + ix
                cur = plsc.load_gather(priv_f, [addr])
                plsc.store_scatter(priv_f, [addr], jnp.maximum(cur, v))
            return carry
        lax.fori_loop(0, NCHUNK // UNR, pass_a, 0)

        def red_a(kk, carry):
            for u in range(2):
                off = pl.multiple_of(kk * (LANES * 2) + u * LANES, LANES)
                acc = priv_f[pl.ds(off, LANES)]
                for l in range(1, LANES):
                    acc = jnp.maximum(acc, priv_f[pl.ds(off + l * N_PAD, LANES)])
                best_f[pl.ds(off, LANES)] = acc
            return carry
        lax.fori_loop(0, NCHUNK // 2, red_a, 0)

        def pass_b(kk, carry):
            for u in range(UNR):
                k = kk * UNR + u
                off = pl.multiple_of(k * LANES, LANES)
                v = sims_v[pl.ds(off, LANES)]
                ix = idxs_v[pl.ds(off, LANES)]
                bj = plsc.load_gather(best_f, [ix])
                cand = jnp.logical_and(v == bj, v >= THRESH)
                rows = k * LANES + lanes
                addr = laneoff + ix
                cur = plsc.load_gather(priv_i, [addr])
                plsc.store_scatter(priv_i, [addr],
                                   jnp.where(cand, jnp.minimum(cur, rows), cur))
            return carry
        lax.fori_loop(0, NCHUNK // UNR, pass_b, 0)

        def red_b(kk, carry):
            for u in range(2):
                off = pl.multiple_of(kk * (LANES * 2) + u * LANES, LANES)
                acc = priv_i[pl.ds(off, LANES)]
                for l in range(1, LANES):
                    acc = jnp.minimum(acc, priv_i[pl.ds(off + l * N_PAD, LANES)])
                best_i[pl.ds(off, LANES)] = acc
            return carry
        lax.fori_loop(0, NCHUNK // 2, red_b, 0)

        def pass_c(kk, carry):
            for u in range(UNR):
                k = kk * UNR + u
                off = pl.multiple_of(k * LANES, LANES)
                ix = idxs_v[pl.ds(off, LANES)]
                w = plsc.load_gather(best_i, [ix])
                rows = k * LANES + lanes
                match_v[pl.ds(off, LANES)] = jnp.where(w == rows, ix, -1)
            return carry
        lax.fori_loop(0, NCHUNK // UNR, pass_c, 0)

        pltpu.sync_copy(match_v, out_hbm)


def _make_sc_match():
    return functools.partial(
        pl.kernel,
        mesh=plsc.VectorSubcoreMesh(core_axis_name="c", subcore_axis_name="s"),
        out_type=jax.ShapeDtypeStruct((N_PAD,), jnp.int32),
        compiler_params=pltpu.CompilerParams(needs_layout_passes=False),
        scratch_types=[
            pltpu.VMEM((N_PAD,), jnp.float32),
            pltpu.VMEM((N_PAD,), jnp.int32),
            pltpu.VMEM((PRIV,), jnp.float32),
            pltpu.VMEM((N_PAD,), jnp.float32),
            pltpu.VMEM((PRIV,), jnp.int32),
            pltpu.VMEM((N_PAD,), jnp.int32),
            pltpu.VMEM((N_PAD,), jnp.int32),
            pltpu.SemaphoreType.DMA,
        ],
    )(_sc_body)


def kernel(detections_t, detections_t1, Wp1, bp1, Wp2, bp2, Wp3, bp3,
           Wa1, ba1, Wa2, ba2, Wr1, br1, Wr2, br2, omega):
    f32 = jnp.float32
    sim, msim, midx = _tc_call(detections_t.T, detections_t1.T, Wp1.T, bp1,
                               Wp2, bp2, Wp3.T, bp3, Wr1, br1, Wr2.T, br2,
                               omega)

    matches = _make_sc_match()(msim, midx, jnp.asarray(_INITF_NP),
                               jnp.asarray(_INITI_NP))
    return matches[:N_T], sim
